# baseline (speedup 1.0000x reference)
import functools

import jax
import jax.numpy as jnp
from jax import lax
from jax.experimental import pallas as pl
from jax.experimental.pallas import tpu as pltpu
from jax.experimental.pallas import tpu_sc as plsc

HIDDEN = 64
SCALE = 8.0
NC = 2
NS = 16
NW = NC * NS
B = 4096 * 200
CH = 128
NSPLIT = 2
BS = B // NSPLIT
BPW = BS // NW
NCHUNK = BPW // CH
NBUF = 4


def _sc_embed(x_hbm, table_hbm, out_hbm, idx_v, rows_v, gsem, osem):
    wid = lax.axis_index("s") * NC + lax.axis_index("c")
    pltpu.sync_copy(x_hbm.at[pl.ds(wid * NCHUNK, NCHUNK)], idx_v)
    out_base = wid * BPW

    def start_gather(g, slot):
        pltpu.async_copy(table_hbm.at[idx_v.at[g]], rows_v.at[slot],
                         gsem.at[slot])

    def wait_gather(slot):
        pltpu.make_async_copy(table_hbm.at[idx_v.at[0]], rows_v.at[slot],
                              gsem.at[slot]).wait()

    def start_out(g, slot):
        pltpu.async_copy(rows_v.at[slot],
                         out_hbm.at[pl.ds(out_base + g * CH, CH)],
                         osem.at[slot])

    def wait_out(slot):
        pltpu.make_async_copy(rows_v.at[slot],
                              out_hbm.at[pl.ds(out_base, CH)],
                              osem.at[slot]).wait()

    def scale(slot):
        def rows4(i, c2):
            for rr in range(4):
                for c in range(HIDDEN // 16):
                    sl = pl.ds(c * 16, 16)
                    rows_v[slot, 4 * i + rr, sl] = (
                        rows_v[slot, 4 * i + rr, sl] * SCALE)
            return c2
        lax.fori_loop(0, CH // 4, rows4, 0)

    start_gather(0, 0)
    for b in range(NBUF):
        wait_gather(b)
        if b < NBUF - 1:
            start_gather(b + 1, b + 1)
        else:
            wait_out(0)
            start_gather(NBUF, 0)
        scale(b)
        start_out(b, b)

    def group(go, c2):
        for b in range(NBUF):
            g = go + b
            slot = b
            nslot = (b + 1) % NBUF
            wait_gather(slot)
            if b < NBUF - 1:
                wait_out(nslot)
                start_gather(g + 1, nslot)
            else:
                @pl.when(go + NBUF < NCHUNK)
                def _():
                    wait_out(nslot)
                    start_gather(g + 1, nslot)
            scale(slot)
            start_out(g, slot)
        return c2

    lax.fori_loop(1, NCHUNK // NBUF, lambda i, c: group(i * NBUF, c), 0)

    for b in range(NBUF):
        wait_out(b)


@jax.jit
def kernel(x, table):
    mesh = plsc.VectorSubcoreMesh(core_axis_name="c", subcore_axis_name="s")
    call = pl.kernel(
        _sc_embed,
        out_type=jax.ShapeDtypeStruct((BS, HIDDEN), jnp.float32),
        mesh=mesh,
        compiler_params=pltpu.CompilerParams(use_tc_tiling_on_sc=False),
        scratch_types=[
            pltpu.VMEM((NCHUNK, CH), jnp.int32),
            pltpu.VMEM((NBUF, CH, HIDDEN), jnp.float32),
            pltpu.SemaphoreType.DMA((NBUF,)),
            pltpu.SemaphoreType.DMA((NBUF,)),
        ],
    )
    rows_per_split = x.shape[0] // NSPLIT
    halves = []
    for s in range(NSPLIT):
        xs = x[s * rows_per_split:(s + 1) * rows_per_split]
        o = call(xs.reshape(BS // CH, CH), table)
        halves.append(o.reshape(rows_per_split, x.shape[1], HIDDEN))
    return jnp.concatenate(halves, axis=0)


# speedup vs baseline: 1.3140x; 1.3140x over previous
import functools

import jax
import jax.numpy as jnp
from jax import lax
from jax.experimental import pallas as pl
from jax.experimental.pallas import tpu as pltpu
from jax.experimental.pallas import tpu_sc as plsc

HIDDEN = 64
SCALE = 8.0
NC = 2
NS = 16
NW = NC * NS
B = 4096 * 200
CH = 128
PADW = 128
BPW = B // NW
NCHUNK = BPW // CH
NBUF = 4


def _sc_embed(x_hbm, table_hbm, out_hbm, idx_v, rows_v, gsem, osem):
    wid = lax.axis_index("s") * NC + lax.axis_index("c")
    pltpu.sync_copy(x_hbm.at[pl.ds(wid * NCHUNK, NCHUNK)], idx_v)
    out_base = wid * BPW

    def start_gather(g, slot):
        pltpu.async_copy(table_hbm.at[idx_v.at[g]], rows_v.at[slot],
                         gsem.at[slot])

    def wait_gather(slot):
        pltpu.make_async_copy(table_hbm.at[idx_v.at[0]], rows_v.at[slot],
                              gsem.at[slot]).wait()

    def start_out(g, slot):
        pltpu.async_copy(rows_v.at[slot],
                         out_hbm.at[pl.ds(out_base + g * CH, CH),
                                    pl.ds(0, HIDDEN)],
                         osem.at[slot])

    def wait_out(slot):
        pltpu.make_async_copy(rows_v.at[slot],
                              out_hbm.at[pl.ds(out_base, CH),
                                         pl.ds(0, HIDDEN)],
                              osem.at[slot]).wait()

    def scale(slot):
        def rows4(i, c2):
            for rr in range(4):
                for c in range(HIDDEN // 16):
                    sl = pl.ds(c * 16, 16)
                    rows_v[slot, 4 * i + rr, sl] = (
                        rows_v[slot, 4 * i + rr, sl] * SCALE)
            return c2
        lax.fori_loop(0, CH // 4, rows4, 0)

    start_gather(0, 0)
    for b in range(NBUF):
        wait_gather(b)
        if b < NBUF - 1:
            start_gather(b + 1, b + 1)
        else:
            wait_out(0)
            start_gather(NBUF, 0)
        scale(b)
        start_out(b, b)

    def group(go, c2):
        for b in range(NBUF):
            g = go + b
            slot = b
            nslot = (b + 1) % NBUF
            wait_gather(slot)
            if b < NBUF - 1:
                wait_out(nslot)
                start_gather(g + 1, nslot)
            else:
                @pl.when(go + NBUF < NCHUNK)
                def _():
                    wait_out(nslot)
                    start_gather(g + 1, nslot)
            scale(slot)
            start_out(g, slot)
        return c2

    lax.fori_loop(1, NCHUNK // NBUF, lambda i, c: group(i * NBUF, c), 0)

    for b in range(NBUF):
        wait_out(b)


@jax.jit
def kernel(x, table):
    x_flat = x.reshape(B // CH, CH)
    mesh = plsc.VectorSubcoreMesh(core_axis_name="c", subcore_axis_name="s")
    out = pl.kernel(
        _sc_embed,
        out_type=jax.ShapeDtypeStruct((B, PADW), jnp.float32),
        mesh=mesh,
        compiler_params=pltpu.CompilerParams(use_tc_tiling_on_sc=False),
        scratch_types=[
            pltpu.VMEM((NCHUNK, CH), jnp.int32),
            pltpu.VMEM((NBUF, CH, HIDDEN), jnp.float32),
            pltpu.SemaphoreType.DMA((NBUF,)),
            pltpu.SemaphoreType.DMA((NBUF,)),
        ],
    )(x_flat, table)
    return out[:, :HIDDEN].reshape(x.shape[0], x.shape[1], HIDDEN)
